# baseline (speedup 1.0000x reference)
import functools

import jax
import jax.numpy as jnp
from jax import lax
from jax.experimental import pallas as pl
from jax.experimental.pallas import tpu as pltpu
from jax.experimental.pallas import tpu_sc as plsc

N_NODES = 10000
N_EDGES = 320000
D = 128

NC = 2
NS = 16
NW = NC * NS

EDGES_PER_TILE = N_EDGES // NW
CHUNK = 40
G_PIPE = 2
NCHUNK = 250
NPAIR = NCHUNK // 2
EDGES_PER_TILE_P = NCHUNK * CHUNK
N_PAD = 10112
ROWS_PER_TILE = N_PAD // NS
PAD_ROW = N_NODES


def _sc_aggregate(x, edge_index, zeros):
    mesh = plsc.VectorSubcoreMesh(core_axis_name="c", subcore_axis_name="s",
                                  num_cores=NC, num_subcores=NS)

    G = G_PIPE
    NJ = 2 * G
    scratch = (
        [pltpu.VMEM((2, CHUNK), jnp.int32)] * (2 * G)
        + [pltpu.VMEM((CHUNK, D), jnp.float32)] * (2 * NJ)
        + [pltpu.VMEM_SHARED((N_PAD, D), jnp.float32)]
        + [pltpu.SemaphoreType.DMA] * (2 * G + 4 * NJ))

    @functools.partial(
        pl.kernel,
        out_type=jax.ShapeDtypeStruct((NC, N_PAD, D), jnp.float32),
        mesh=mesh,
        scratch_types=scratch,
    )
    def agg_kernel(x_hbm, s_hbm, r_hbm, zeros_hbm, out_hbm, *sc):
        cid = lax.axis_index("c")
        sid = lax.axis_index("s")
        wid = cid * NS + sid
        s2, r2 = sc[0:G], sc[G:2 * G]
        o = 2 * G
        rows_a, rows_b = sc[o:o + NJ], sc[o + NJ:o + 2 * NJ]
        acc_sh = sc[o + 2 * NJ]
        sems = sc[o + 2 * NJ + 1:]
        isem, rsem = sems[0:G], sems[G:2 * G]
        gsem_a, gsem_b = sems[2 * G:2 * G + NJ], sems[2 * G + NJ:2 * G + 2 * NJ]
        ssem_a = sems[2 * G + 2 * NJ:2 * G + 3 * NJ]
        ssem_b = sems[2 * G + 3 * NJ:2 * G + 4 * NJ]
        pbase = wid * NPAIR

        rbase = sid * ROWS_PER_TILE
        pltpu.sync_copy(zeros_hbm.at[pl.ds(rbase, ROWS_PER_TILE)],
                        acc_sh.at[pl.ds(rbase, ROWS_PER_TILE)])
        plsc.subcore_barrier()

        def group(base_pair, n_pairs):
            idescs = []
            for k in range(n_pairs):
                idescs.append(pltpu.async_copy(
                    s_hbm.at[base_pair + k], s2[k], isem[k]))
                idescs.append(pltpu.async_copy(
                    r_hbm.at[base_pair + k], r2[k], rsem[k]))
            gdescs = []
            for k in range(n_pairs):
                idescs[2 * k].wait()
                idescs[2 * k + 1].wait()
                for j in range(2):
                    c = 2 * k + j
                    gdescs.append(pltpu.async_copy(
                        x_hbm.at[s2[k].at[j]], rows_a[c], gsem_a[c]))
                    gdescs.append(pltpu.async_copy(
                        x_hbm.at[r2[k].at[j]], rows_b[c], gsem_b[c]))
            sdescs = []
            for k in range(n_pairs):
                for j in range(2):
                    c = 2 * k + j
                    gdescs[2 * c].wait()
                    sdescs.append(pltpu.async_copy(
                        rows_a[c], acc_sh.at[r2[k].at[j]], ssem_a[c], add=True))
                    gdescs[2 * c + 1].wait()
                    sdescs.append(pltpu.async_copy(
                        rows_b[c], acc_sh.at[s2[k].at[j]], ssem_b[c], add=True))
            for d in sdescs:
                d.wait()

        @pl.loop(0, NPAIR // G)
        def _(g):
            group(pbase + G * g, G)

        if NPAIR % G:
            group(pbase + NPAIR - NPAIR % G, NPAIR % G)

        plsc.subcore_barrier()
        pltpu.sync_copy(acc_sh.at[pl.ds(rbase, ROWS_PER_TILE)],
                        out_hbm.at[cid, pl.ds(rbase, ROWS_PER_TILE)])

    s_pairs = edge_index[0].reshape(NW * NPAIR, 2, CHUNK)
    r_pairs = edge_index[1].reshape(NW * NPAIR, 2, CHUNK)
    return agg_kernel(x, s_pairs, r_pairs, zeros)


BLK = 1000


def _gelu_exact(v):
    return 0.5 * v * (1.0 + lax.erf(v * 0.7071067811865476))


def _mlp_body(p_ref, w1_ref, b1_ref, w2_ref, b2_ref, w3_ref, b3_ref,
              g_ref, bt_ref, o_ref):
    agg = p_ref[0] + p_ref[1]
    h = jnp.dot(agg, w1_ref[:], preferred_element_type=jnp.float32) + b1_ref[:]
    h = _gelu_exact(h)
    h = jnp.dot(h, w2_ref[:], preferred_element_type=jnp.float32) + b2_ref[:]
    h = _gelu_exact(h)
    o = jnp.dot(h, w3_ref[:], preferred_element_type=jnp.float32) + b3_ref[:]
    mu = jnp.mean(o, axis=-1, keepdims=True)
    var = jnp.mean((o - mu) ** 2, axis=-1, keepdims=True)
    o_ref[:] = (o - mu) / jnp.sqrt(var + 1e-5) * g_ref[:] + bt_ref[:]


def _tc_mlp(parts, W1, b1, W2, b2, W3, b3, gamma, beta):
    vec = pl.BlockSpec((1, D), lambda i: (0, 0))
    mat = pl.BlockSpec((D, D), lambda i: (0, 0))
    return pl.pallas_call(
        _mlp_body,
        grid=(N_NODES // BLK,),
        in_specs=[pl.BlockSpec((NC, BLK, D), lambda i: (0, i, 0)),
                  mat, vec, mat, vec, mat, vec, vec, vec],
        out_specs=pl.BlockSpec((BLK, D), lambda i: (i, 0)),
        out_shape=jax.ShapeDtypeStruct((N_NODES, D), jnp.float32),
    )(parts, W1, b1.reshape(1, D), W2, b2.reshape(1, D),
      W3, b3.reshape(1, D), gamma.reshape(1, D), beta.reshape(1, D))


def kernel(x, edge_index, W1, b1, W2, b2, W3, b3, gamma, beta):
    ei = edge_index.astype(jnp.int32)
    zeros = jnp.zeros((N_PAD, D), jnp.float32)
    parts = _sc_aggregate(x, ei, zeros)
    return _tc_mlp(parts, W1, b1, W2, b2, W3, b3, gamma, beta)


# speedup vs baseline: 1.0923x; 1.0923x over previous
import functools

import jax
import jax.numpy as jnp
from jax import lax
from jax.experimental import pallas as pl
from jax.experimental.pallas import tpu as pltpu
from jax.experimental.pallas import tpu_sc as plsc

N_NODES = 10000
N_EDGES = 320000
D = 128

NC = 2
NS = 16
NW = NC * NS

EDGES_PER_TILE = N_EDGES // NW
CHUNK = 40
G_PIPE = 4
NCHUNK = 250
EDGES_PER_TILE_P = NCHUNK * CHUNK
N_PAD = 10112
ROWS_PER_TILE = N_PAD // NS
PAD_ROW = N_NODES


def _sc_aggregate(x, edge_index, zeros):
    mesh = plsc.VectorSubcoreMesh(core_axis_name="c", subcore_axis_name="s",
                                  num_cores=NC, num_subcores=NS)

    G = G_PIPE
    scratch = (
        [pltpu.VMEM((CHUNK,), jnp.int32)] * (2 * G)
        + [pltpu.VMEM((CHUNK, D), jnp.float32)] * (2 * G)
        + [pltpu.VMEM_SHARED((N_PAD, D), jnp.float32)]
        + [pltpu.SemaphoreType.DMA] * (6 * G))

    @functools.partial(
        pl.kernel,
        out_type=jax.ShapeDtypeStruct((NC, N_PAD, D), jnp.float32),
        mesh=mesh,
        scratch_types=scratch,
    )
    def agg_kernel(x_hbm, s_hbm, r_hbm, zeros_hbm, out_hbm, *sc):
        cid = lax.axis_index("c")
        sid = lax.axis_index("s")
        wid = cid * NS + sid
        si, ri = sc[0:G], sc[G:2 * G]
        rows_a, rows_b = sc[2 * G:3 * G], sc[3 * G:4 * G]
        acc_sh = sc[4 * G]
        sems = sc[4 * G + 1:]
        gsem_a, gsem_b = sems[0:G], sems[G:2 * G]
        isem, rsem = sems[2 * G:3 * G], sems[3 * G:4 * G]
        ssem_a, ssem_b = sems[4 * G:5 * G], sems[5 * G:6 * G]
        cbase = wid * NCHUNK

        rbase = sid * ROWS_PER_TILE
        pltpu.sync_copy(zeros_hbm.at[pl.ds(rbase, ROWS_PER_TILE)],
                        acc_sh.at[pl.ds(rbase, ROWS_PER_TILE)])
        plsc.subcore_barrier()

        def group(base_chunk, n_chunks):
            idescs = []
            for j in range(n_chunks):
                off = base_chunk * CHUNK + j * CHUNK
                idescs.append(pltpu.async_copy(
                    s_hbm.at[pl.ds(off, CHUNK)], si[j], isem[j]))
                idescs.append(pltpu.async_copy(
                    r_hbm.at[pl.ds(off, CHUNK)], ri[j], rsem[j]))
            gdescs = []
            for j in range(n_chunks):
                idescs[2 * j].wait()
                idescs[2 * j + 1].wait()
                gdescs.append(pltpu.async_copy(
                    x_hbm.at[si[j]], rows_a[j], gsem_a[j]))
                gdescs.append(pltpu.async_copy(
                    x_hbm.at[ri[j]], rows_b[j], gsem_b[j]))
            sdescs = []
            for j in range(n_chunks):
                gdescs[2 * j].wait()
                sdescs.append(pltpu.async_copy(
                    rows_a[j], acc_sh.at[ri[j]], ssem_a[j], add=True))
                gdescs[2 * j + 1].wait()
                sdescs.append(pltpu.async_copy(
                    rows_b[j], acc_sh.at[si[j]], ssem_b[j], add=True))
            for d in sdescs:
                d.wait()

        @pl.loop(0, NCHUNK // G)
        def _(g):
            group(cbase + G * g, G)

        if NCHUNK % G:
            group(cbase + NCHUNK - NCHUNK % G, NCHUNK % G)

        plsc.subcore_barrier()
        pltpu.sync_copy(acc_sh.at[pl.ds(rbase, ROWS_PER_TILE)],
                        out_hbm.at[cid, pl.ds(rbase, ROWS_PER_TILE)])

    if EDGES_PER_TILE_P == EDGES_PER_TILE:
        return agg_kernel(x, edge_index[0], edge_index[1], zeros)
    npad = EDGES_PER_TILE_P - EDGES_PER_TILE
    s_p = jnp.pad(edge_index[0].reshape(NW, EDGES_PER_TILE),
                  ((0, 0), (0, npad))).reshape(-1)
    pad_rows = PAD_ROW + (jnp.arange(npad, dtype=jnp.int32) % (N_PAD - N_NODES))
    r_p = jnp.concatenate(
        [edge_index[1].reshape(NW, EDGES_PER_TILE),
         jnp.broadcast_to(pad_rows, (NW, npad))], axis=1).reshape(-1)
    x_p = jnp.pad(x, ((0, N_PAD - N_NODES), (0, 0)))
    return agg_kernel(x_p, s_p, r_p, zeros)


BLK = 1000


def _gelu_exact(v):
    return 0.5 * v * (1.0 + lax.erf(v * 0.7071067811865476))


def _mlp_body(p_ref, w1_ref, b1_ref, w2_ref, b2_ref, w3_ref, b3_ref,
              g_ref, bt_ref, o_ref):
    agg = p_ref[0] + p_ref[1]
    h = jnp.dot(agg, w1_ref[:], preferred_element_type=jnp.float32) + b1_ref[:]
    h = _gelu_exact(h)
    h = jnp.dot(h, w2_ref[:], preferred_element_type=jnp.float32) + b2_ref[:]
    h = _gelu_exact(h)
    o = jnp.dot(h, w3_ref[:], preferred_element_type=jnp.float32) + b3_ref[:]
    mu = jnp.mean(o, axis=-1, keepdims=True)
    var = jnp.mean((o - mu) ** 2, axis=-1, keepdims=True)
    o_ref[:] = (o - mu) / jnp.sqrt(var + 1e-5) * g_ref[:] + bt_ref[:]


def _tc_mlp(parts, W1, b1, W2, b2, W3, b3, gamma, beta):
    vec = pl.BlockSpec((1, D), lambda i: (0, 0))
    mat = pl.BlockSpec((D, D), lambda i: (0, 0))
    return pl.pallas_call(
        _mlp_body,
        grid=(N_NODES // BLK,),
        in_specs=[pl.BlockSpec((NC, BLK, D), lambda i: (0, i, 0)),
                  mat, vec, mat, vec, mat, vec, vec, vec],
        out_specs=pl.BlockSpec((BLK, D), lambda i: (i, 0)),
        out_shape=jax.ShapeDtypeStruct((N_NODES, D), jnp.float32),
    )(parts, W1, b1.reshape(1, D), W2, b2.reshape(1, D),
      W3, b3.reshape(1, D), gamma.reshape(1, D), beta.reshape(1, D))


def kernel(x, edge_index, W1, b1, W2, b2, W3, b3, gamma, beta):
    ei = edge_index.astype(jnp.int32)
    zeros = jnp.zeros((N_PAD, D), jnp.float32)
    parts = _sc_aggregate(x, ei, zeros)
    return _tc_mlp(parts, W1, b1, W2, b2, W3, b3, gamma, beta)
